# baseline (speedup 1.0000x reference)
import functools

import jax
import jax.numpy as jnp
from jax import lax
from jax.experimental import pallas as pl
from jax.experimental.pallas import tpu as pltpu
from jax.experimental.pallas import tpu_sc as plsc

_N = 10000
_NPAD = 10240
_D = 128
_NC = 2
_NS = 16
_NW = _NC * _NS
_CH = 96
_BLK = 1024
_ROWS_PER_SUB = _NPAD // _NS
_ZCH = 64
_ROW_CHUNKS = _ROWS_PER_SUB // _ZCH
_EPS = 1e-16
_SLOPE = 0.2


def _tc_first_body(x_ref, w_ref, ss_ref, sd_ref, h_ref, ts_ref, td_ref):
    h = jnp.dot(x_ref[...], w_ref[...], preferred_element_type=jnp.float32)
    h_ref[...] = h
    ts_ref[...] = jnp.dot(h, ss_ref[...], preferred_element_type=jnp.float32)
    td_ref[...] = jnp.dot(h, sd_ref[...], preferred_element_type=jnp.float32)


def _epilogue(acc0, acc1, den0, den1, b):
    f = (acc0 + acc1) / (den0 + den1 + _EPS) + b
    return jnp.maximum(f, 0.0)


def _tc_mid_body(a0_ref, a1_ref, d0_ref, d1_ref, b_ref, w_ref, ss_ref, sd_ref,
                 h_ref, ts_ref, td_ref):
    f = _epilogue(a0_ref[...], a1_ref[...], d0_ref[...], d1_ref[...], b_ref[...])
    h = jnp.dot(f, w_ref[...], preferred_element_type=jnp.float32)
    h_ref[...] = h
    ts_ref[...] = jnp.dot(h, ss_ref[...], preferred_element_type=jnp.float32)
    td_ref[...] = jnp.dot(h, sd_ref[...], preferred_element_type=jnp.float32)


def _tc_last_body(a0_ref, a1_ref, d0_ref, d1_ref, b_ref, w_ref, lb_ref, o_ref):
    f = _epilogue(a0_ref[...], a1_ref[...], d0_ref[...], d1_ref[...], b_ref[...])
    o = jnp.dot(f, w_ref[...], preferred_element_type=jnp.float32) + lb_ref[...]
    o_ref[...] = jnp.maximum(o, 0.0)


def _row_spec():
    return pl.BlockSpec((_BLK, _D), lambda i: (i, 0))


def _full_spec(r):
    return pl.BlockSpec((r, _D), lambda i: (0, 0))


_GRID = (_NPAD // _BLK,)
_3OUT = [jax.ShapeDtypeStruct((_NPAD, _D), jnp.float32)] * 3

_tc_first = pl.pallas_call(
    _tc_first_body,
    grid=_GRID,
    in_specs=[_row_spec(), _full_spec(_D), _full_spec(_D), _full_spec(_D)],
    out_specs=[_row_spec()] * 3,
    out_shape=_3OUT,
)

_tc_mid = pl.pallas_call(
    _tc_mid_body,
    grid=_GRID,
    in_specs=[_row_spec(), _row_spec(), _row_spec(), _row_spec(),
              _full_spec(1), _full_spec(_D), _full_spec(_D), _full_spec(_D)],
    out_specs=[_row_spec()] * 3,
    out_shape=_3OUT,
)

_tc_last = pl.pallas_call(
    _tc_last_body,
    grid=_GRID,
    in_specs=[_row_spec(), _row_spec(), _row_spec(), _row_spec(),
              _full_spec(1), _full_spec(_D), _full_spec(1)],
    out_specs=_row_spec(),
    out_shape=jax.ShapeDtypeStruct((_NPAD, _D), jnp.float32),
)


def _edge_weight(s16, d16):
    er = s16 + d16
    er = jnp.where(er >= 0.0, er, er * _SLOPE)
    return jnp.exp(er)


def _make_sc_edge_pass(ept):
    nchunk = ept // _CH
    mesh = plsc.VectorSubcoreMesh(core_axis_name="c", subcore_axis_name="s")

    @functools.partial(
        pl.kernel,
        out_type=(jax.ShapeDtypeStruct((_NC, _NPAD, _D), jnp.float32),
                  jax.ShapeDtypeStruct((_NC, _NPAD, _D), jnp.float32)),
        mesh=mesh,
        scratch_types=[
            pltpu.VMEM((_CH,), jnp.int32),
            pltpu.VMEM((_CH,), jnp.int32),
            pltpu.VMEM((_CH, _D), jnp.float32),
            pltpu.VMEM((_CH, _D), jnp.float32),
            pltpu.VMEM((_CH, _D), jnp.float32),
            pltpu.VMEM_SHARED((_NPAD, _D), jnp.float32),
            pltpu.SemaphoreType.DMA,
            pltpu.SemaphoreType.DMA,
            pltpu.SemaphoreType.DMA,
        ],
    )
    def sc_edge_pass(h_hbm, ts_hbm, td_hbm, src_hbm, dst_hbm,
                     acc_out, den_out,
                     src_v, dst_v, ts_v, td_v, h_v,
                     acc_sh, sem0, sem1, sem2):
        cid = lax.axis_index("c")
        sid = lax.axis_index("s")
        wid = sid * _NC + cid
        ebase = wid * ept

        def zero_own_rows():
            def zrow(r, _):
                for j in range(_D // 16):
                    ts_v[r, pl.ds(j * 16, 16)] = jnp.zeros((16,), jnp.float32)
                return 0
            lax.fori_loop(0, _ZCH, zrow, 0)

            def zcopy(i, _):
                r0 = sid * _ROWS_PER_SUB + i * _ZCH
                pltpu.sync_copy(ts_v.at[pl.ds(0, _ZCH), :],
                                acc_sh.at[pl.ds(r0, _ZCH), :])
                return 0
            lax.fori_loop(0, _ROW_CHUNKS, zcopy, 0)

        def writeback(out_ref):
            def wloop(i, _):
                r0 = sid * _ROWS_PER_SUB + i * _ZCH
                pltpu.sync_copy(acc_sh.at[pl.ds(r0, _ZCH), :],
                                h_v.at[pl.ds(0, _ZCH), :])
                pltpu.sync_copy(h_v.at[pl.ds(0, _ZCH), :],
                                out_ref.at[cid, pl.ds(r0, _ZCH), :])
                return 0
            lax.fori_loop(0, _ROW_CHUNKS, wloop, 0)

        def edge_sweep(with_h):
            def chunk_body(ci, _):
                base = ebase + ci * _CH
                pltpu.sync_copy(src_hbm.at[pl.ds(base, _CH)], src_v)
                pltpu.sync_copy(dst_hbm.at[pl.ds(base, _CH)], dst_v)
                ca = pltpu.async_copy(ts_hbm.at[src_v], ts_v, sem0)
                cb = pltpu.async_copy(td_hbm.at[dst_v], td_v, sem1)
                if with_h:
                    cc = pltpu.async_copy(h_hbm.at[src_v], h_v, sem2)
                ca.wait()
                cb.wait()
                if with_h:
                    cc.wait()

                def edge_body(e, _):
                    for j in range(_D // 16):
                        sl = pl.ds(j * 16, 16)
                        p = _edge_weight(ts_v[e, sl], td_v[e, sl])
                        if with_h:
                            p = p * h_v[e, sl]
                        h_v[e, sl] = p
                    return 0
                lax.fori_loop(0, _CH, edge_body, 0, unroll=4)

                pltpu.sync_copy(h_v, acc_sh.at[dst_v], add=True)
                return 0
            lax.fori_loop(0, nchunk, chunk_body, 0)

        zero_own_rows()
        plsc.subcore_barrier()
        edge_sweep(with_h=True)
        plsc.subcore_barrier()
        writeback(acc_out)
        zero_own_rows()
        plsc.subcore_barrier()
        edge_sweep(with_h=False)
        plsc.subcore_barrier()
        writeback(den_out)

    return sc_edge_pass


def _expand_table(a):
    heads, ch = a.shape
    if heads == 8:
        s = (a[:, :, None, None]
             * jnp.eye(8, dtype=a.dtype)[:, None, :, None]
             * jnp.ones((1, 1, 1, 16), a.dtype))
        return s.reshape(128, 128)
    return jnp.tile(a.reshape(128, 1), (1, 128))


def kernel(x, edge_attr, W1, as1, ad1, b1, W2, as2, ad2, b2, W3, as3, ad3, b3,
           W4, as4, ad4, b4, W5, as5, ad5, b5, linW, linb, edge_index):
    n = x.shape[0]
    e = edge_index.shape[1]
    e_total = e + n
    ept = -(-e_total // _NW)
    ept = -(-ept // _CH) * _CH
    epad = ept * _NW

    ei = edge_index.astype(jnp.int32)
    loop = jnp.arange(n, dtype=jnp.int32)
    padv = jnp.full((epad - e_total,), n, jnp.int32)
    src = jnp.concatenate([ei[0], loop, padv])
    dst = jnp.concatenate([ei[1], loop, padv])

    xp = jnp.pad(x, ((0, _NPAD - n), (0, 0)))

    sc_edge = _make_sc_edge_pass(ept)

    h, t_s, t_d = _tc_first(xp, W1, _expand_table(as1), _expand_table(ad1))
    acc, den = sc_edge(h, t_s, t_d, src, dst)
    for (W, a_s, a_d, b) in ((W2, as2, ad2, b1), (W3, as3, ad3, b2),
                             (W4, as4, ad4, b3), (W5, as5, ad5, b4)):
        h, t_s, t_d = _tc_mid(acc[0], acc[1], den[0], den[1],
                              b.reshape(1, _D), W, _expand_table(a_s),
                              _expand_table(a_d))
        acc, den = sc_edge(h, t_s, t_d, src, dst)
    out = _tc_last(acc[0], acc[1], den[0], den[1], b5.reshape(1, _D),
                   linW, linb.reshape(1, _D))
    return out[:n]


# speedup vs baseline: 2.7722x; 2.7722x over previous
import functools

import jax
import jax.numpy as jnp
from jax import lax
from jax.experimental import pallas as pl
from jax.experimental.pallas import tpu as pltpu
from jax.experimental.pallas import tpu_sc as plsc

_N = 10000
_NPAD = 10240
_D = 128
_NC = 2
_NS = 16
_NW = _NC * _NS
_CH = 96
_BLK = 1024
_ROWS_PER_SUB = _NPAD // _NS
_ZCH = 64
_ROW_CHUNKS = _ROWS_PER_SUB // _ZCH
_EPS = 1e-16
_SLOPE = 0.2


def _tc_first_body(x_ref, w_ref, ss_ref, sd_ref, h_ref, ts_ref, td_ref):
    h = jnp.dot(x_ref[...], w_ref[...], preferred_element_type=jnp.float32)
    h_ref[...] = h
    ts_ref[...] = jnp.dot(h, ss_ref[...], preferred_element_type=jnp.float32)
    td_ref[...] = jnp.dot(h, sd_ref[...], preferred_element_type=jnp.float32)


def _epilogue(acc0, acc1, den0, den1, b):
    f = (acc0 + acc1) / (den0 + den1 + _EPS) + b
    return jnp.maximum(f, 0.0)


def _tc_mid_body(a0_ref, a1_ref, d0_ref, d1_ref, b_ref, w_ref, ss_ref, sd_ref,
                 h_ref, ts_ref, td_ref):
    f = _epilogue(a0_ref[...], a1_ref[...], d0_ref[...], d1_ref[...], b_ref[...])
    h = jnp.dot(f, w_ref[...], preferred_element_type=jnp.float32)
    h_ref[...] = h
    ts_ref[...] = jnp.dot(h, ss_ref[...], preferred_element_type=jnp.float32)
    td_ref[...] = jnp.dot(h, sd_ref[...], preferred_element_type=jnp.float32)


def _tc_last_body(a0_ref, a1_ref, d0_ref, d1_ref, b_ref, w_ref, lb_ref, o_ref):
    f = _epilogue(a0_ref[...], a1_ref[...], d0_ref[...], d1_ref[...], b_ref[...])
    o = jnp.dot(f, w_ref[...], preferred_element_type=jnp.float32) + lb_ref[...]
    o_ref[...] = jnp.maximum(o, 0.0)


def _row_spec():
    return pl.BlockSpec((_BLK, _D), lambda i: (i, 0))


def _full_spec(r):
    return pl.BlockSpec((r, _D), lambda i: (0, 0))


_GRID = (_NPAD // _BLK,)
_3OUT = [jax.ShapeDtypeStruct((_NPAD, _D), jnp.float32)] * 3

_tc_first = pl.pallas_call(
    _tc_first_body,
    grid=_GRID,
    in_specs=[_row_spec(), _full_spec(_D), _full_spec(_D), _full_spec(_D)],
    out_specs=[_row_spec()] * 3,
    out_shape=_3OUT,
)

_tc_mid = pl.pallas_call(
    _tc_mid_body,
    grid=_GRID,
    in_specs=[_row_spec(), _row_spec(), _row_spec(), _row_spec(),
              _full_spec(1), _full_spec(_D), _full_spec(_D), _full_spec(_D)],
    out_specs=[_row_spec()] * 3,
    out_shape=_3OUT,
)

_tc_last = pl.pallas_call(
    _tc_last_body,
    grid=_GRID,
    in_specs=[_row_spec(), _row_spec(), _row_spec(), _row_spec(),
              _full_spec(1), _full_spec(_D), _full_spec(1)],
    out_specs=_row_spec(),
    out_shape=jax.ShapeDtypeStruct((_NPAD, _D), jnp.float32),
)


def _edge_weight(s16, d16):
    er = s16 + d16
    er = jnp.where(er >= 0.0, er, er * _SLOPE)
    return jnp.exp(er)


def _make_sc_edge_pass(ept):
    nchunk = ept // _CH
    mesh = plsc.VectorSubcoreMesh(core_axis_name="c", subcore_axis_name="s")

    @functools.partial(
        pl.kernel,
        out_type=(jax.ShapeDtypeStruct((_NC, _NPAD, _D), jnp.float32),
                  jax.ShapeDtypeStruct((_NC, _NPAD, _D), jnp.float32)),
        mesh=mesh,
        scratch_types=[
            pltpu.VMEM((_CH,), jnp.int32),
            pltpu.VMEM((_CH,), jnp.int32),
            pltpu.VMEM((_CH, _D), jnp.float32),
            pltpu.VMEM((_CH, _D), jnp.float32),
            pltpu.VMEM((_CH, _D), jnp.float32),
            pltpu.VMEM_SHARED((_NPAD, _D), jnp.float32),
            pltpu.SemaphoreType.DMA,
            pltpu.SemaphoreType.DMA,
            pltpu.SemaphoreType.DMA,
        ],
    )
    def sc_edge_pass(h_hbm, ts_hbm, td_hbm, src_hbm, dst_hbm,
                     acc_out, den_out,
                     src_v, dst_v, ts_v, td_v, h_v,
                     acc_sh, sem0, sem1, sem2):
        cid = lax.axis_index("c")
        sid = lax.axis_index("s")
        wid = sid * _NC + cid
        ebase = wid * ept

        def zero_own_rows():
            def zrow(r, _):
                for j in range(_D // 16):
                    ts_v[r, pl.ds(j * 16, 16)] = jnp.zeros((16,), jnp.float32)
                return 0
            lax.fori_loop(0, _ZCH, zrow, 0)

            def zcopy(i, _):
                r0 = sid * _ROWS_PER_SUB + i * _ZCH
                pltpu.sync_copy(ts_v.at[pl.ds(0, _ZCH), :],
                                acc_sh.at[pl.ds(r0, _ZCH), :])
                return 0
            lax.fori_loop(0, _ROW_CHUNKS, zcopy, 0)

        def writeback(out_ref):
            def wloop(i, _):
                r0 = sid * _ROWS_PER_SUB + i * _ZCH
                pltpu.sync_copy(acc_sh.at[pl.ds(r0, _ZCH), :],
                                h_v.at[pl.ds(0, _ZCH), :])
                pltpu.sync_copy(h_v.at[pl.ds(0, _ZCH), :],
                                out_ref.at[cid, pl.ds(r0, _ZCH), :])
                return 0
            lax.fori_loop(0, _ROW_CHUNKS, wloop, 0)

        def edge_sweep(with_h):
            def chunk_body(ci, _):
                base = ebase + ci * _CH
                pltpu.sync_copy(src_hbm.at[pl.ds(base, _CH)], src_v)
                pltpu.sync_copy(dst_hbm.at[pl.ds(base, _CH)], dst_v)
                ca = pltpu.async_copy(ts_hbm.at[src_v], ts_v, sem0)
                cb = pltpu.async_copy(td_hbm.at[dst_v], td_v, sem1)
                if with_h:
                    cc = pltpu.async_copy(h_hbm.at[src_v], h_v, sem2)
                ca.wait()
                cb.wait()
                if with_h:
                    cc.wait()

                def edge_body(e, _):
                    for j in range(_D // 16):
                        sl = pl.ds(j * 16, 16)
                        p = _edge_weight(ts_v[e, sl], td_v[e, sl])
                        if with_h:
                            p = p * h_v[e, sl]
                        h_v[e, sl] = p
                    return 0
                lax.fori_loop(0, _CH, edge_body, 0)

                pltpu.sync_copy(h_v, acc_sh.at[dst_v], add=True)
                return 0
            lax.fori_loop(0, nchunk, chunk_body, 0)

        zero_own_rows()
        plsc.subcore_barrier()
        edge_sweep(with_h=True)
        plsc.subcore_barrier()
        writeback(acc_out)
        zero_own_rows()
        plsc.subcore_barrier()
        edge_sweep(with_h=False)
        plsc.subcore_barrier()
        writeback(den_out)

    return sc_edge_pass


def _expand_table(a):
    heads, ch = a.shape
    if heads == 8:
        s = (a[:, :, None, None]
             * jnp.eye(8, dtype=a.dtype)[:, None, :, None]
             * jnp.ones((1, 1, 1, 16), a.dtype))
        return s.reshape(128, 128)
    return jnp.tile(a.reshape(128, 1), (1, 128))


def kernel(x, edge_attr, W1, as1, ad1, b1, W2, as2, ad2, b2, W3, as3, ad3, b3,
           W4, as4, ad4, b4, W5, as5, ad5, b5, linW, linb, edge_index):
    n = x.shape[0]
    e = edge_index.shape[1]
    e_total = e + n
    ept = -(-e_total // _NW)
    ept = -(-ept // _CH) * _CH
    epad = ept * _NW

    ei = edge_index.astype(jnp.int32)
    loop = jnp.arange(n, dtype=jnp.int32)
    padv = jnp.full((epad - e_total,), n, jnp.int32)
    src = jnp.concatenate([ei[0], loop, padv])
    dst = jnp.concatenate([ei[1], loop, padv])

    xp = jnp.pad(x, ((0, _NPAD - n), (0, 0)))

    sc_edge = _make_sc_edge_pass(ept)

    h, t_s, t_d = _tc_first(xp, W1, _expand_table(as1), _expand_table(ad1))
    acc, den = sc_edge(h, t_s, t_d, src, dst)
    for (W, a_s, a_d, b) in ((W2, as2, ad2, b1), (W3, as3, ad3, b2),
                             (W4, as4, ad4, b3), (W5, as5, ad5, b4)):
        h, t_s, t_d = _tc_mid(acc[0], acc[1], den[0], den[1],
                              b.reshape(1, _D), W, _expand_table(a_s),
                              _expand_table(a_d))
        acc, den = sc_edge(h, t_s, t_d, src, dst)
    out = _tc_last(acc[0], acc[1], den[0], den[1], b5.reshape(1, _D),
                   linW, linb.reshape(1, _D))
    return out[:n]


# speedup vs baseline: 3.8434x; 1.3864x over previous
import functools

import jax
import jax.numpy as jnp
from jax import lax
from jax.experimental import pallas as pl
from jax.experimental.pallas import tpu as pltpu
from jax.experimental.pallas import tpu_sc as plsc

_N = 10000
_NPAD = 10112
_D = 128
_NC = 2
_NS = 16
_NW = _NC * _NS
_CH = 64
_ROWS_PER_SUB = _NPAD // _NS
_ROW_PIECES = [(i * 64, 64) for i in range(9)] + [(576, 56)]
_EPS = 1e-16
_SLOPE = 0.2


def _tc_first_body(x_ref, w_ref, ss_ref, sd_ref, h_ref, ts_ref, td_ref):
    h = jnp.dot(x_ref[...], w_ref[...], preferred_element_type=jnp.float32)
    h_ref[...] = h
    ts_ref[...] = jnp.dot(h, ss_ref[...], preferred_element_type=jnp.float32)
    td_ref[...] = jnp.dot(h, sd_ref[...], preferred_element_type=jnp.float32)


def _epilogue(acc0, acc1, den0, den1, b):
    f = (acc0 + acc1) / (den0 + den1 + _EPS) + b
    return jnp.maximum(f, 0.0)


def _tc_mid_body(a0_ref, a1_ref, d0_ref, d1_ref, b_ref, w_ref, ss_ref, sd_ref,
                 h_ref, ts_ref, td_ref):
    f = _epilogue(a0_ref[...], a1_ref[...], d0_ref[...], d1_ref[...], b_ref[...])
    h = jnp.dot(f, w_ref[...], preferred_element_type=jnp.float32)
    h_ref[...] = h
    ts_ref[...] = jnp.dot(h, ss_ref[...], preferred_element_type=jnp.float32)
    td_ref[...] = jnp.dot(h, sd_ref[...], preferred_element_type=jnp.float32)


def _tc_last_body(a0_ref, a1_ref, d0_ref, d1_ref, b_ref, w_ref, lb_ref, o_ref):
    f = _epilogue(a0_ref[...], a1_ref[...], d0_ref[...], d1_ref[...], b_ref[...])
    o = jnp.dot(f, w_ref[...], preferred_element_type=jnp.float32) + lb_ref[...]
    o_ref[...] = jnp.maximum(o, 0.0)


def _row_spec():
    return pl.BlockSpec((_NPAD, _D), lambda: (0, 0))


def _full_spec(r):
    return pl.BlockSpec((r, _D), lambda: (0, 0))


_3OUT = [jax.ShapeDtypeStruct((_NPAD, _D), jnp.float32)] * 3

_tc_first = pl.pallas_call(
    _tc_first_body,
    in_specs=[_row_spec(), _full_spec(_D), _full_spec(_D), _full_spec(_D)],
    out_specs=[_row_spec()] * 3,
    out_shape=_3OUT,
)

_tc_mid = pl.pallas_call(
    _tc_mid_body,
    in_specs=[_row_spec(), _row_spec(), _row_spec(), _row_spec(),
              _full_spec(1), _full_spec(_D), _full_spec(_D), _full_spec(_D)],
    out_specs=[_row_spec()] * 3,
    out_shape=_3OUT,
)

_tc_last = pl.pallas_call(
    _tc_last_body,
    in_specs=[_row_spec(), _row_spec(), _row_spec(), _row_spec(),
              _full_spec(1), _full_spec(_D), _full_spec(1)],
    out_specs=_row_spec(),
    out_shape=jax.ShapeDtypeStruct((_NPAD, _D), jnp.float32),
)


def _edge_weight(s16, d16):
    er = s16 + d16
    er = jnp.where(er >= 0.0, er, er * _SLOPE)
    return jnp.exp(er)


def _make_sc_edge_pass(ept):
    nchunk = ept // _CH
    npair = nchunk // 2
    mesh = plsc.VectorSubcoreMesh(core_axis_name="c", subcore_axis_name="s")

    @functools.partial(
        pl.kernel,
        out_type=(jax.ShapeDtypeStruct((_NC, _NPAD, _D), jnp.float32),
                  jax.ShapeDtypeStruct((_NC, _NPAD, _D), jnp.float32)),
        mesh=mesh,
        scratch_types=[
            pltpu.VMEM((_CH,), jnp.int32),
            pltpu.VMEM((_CH,), jnp.int32),
            pltpu.VMEM((_CH, _D), jnp.float32),
            pltpu.VMEM((_CH, _D), jnp.float32),
            pltpu.VMEM((_CH, _D), jnp.float32),
            pltpu.VMEM((_CH,), jnp.int32),
            pltpu.VMEM((_CH,), jnp.int32),
            pltpu.VMEM((_CH, _D), jnp.float32),
            pltpu.VMEM((_CH, _D), jnp.float32),
            pltpu.VMEM((_CH, _D), jnp.float32),
            pltpu.VMEM_SHARED((_NPAD, _D), jnp.float32),
            pltpu.SemaphoreType.DMA,
            pltpu.SemaphoreType.DMA,
            pltpu.SemaphoreType.DMA,
            pltpu.SemaphoreType.DMA,
            pltpu.SemaphoreType.DMA,
            pltpu.SemaphoreType.DMA,
        ],
    )
    def sc_edge_pass(h_hbm, ts_hbm, td_hbm, src_hbm, dst_hbm,
                     acc_out, den_out,
                     src0, dst0, ts0, td0, h0,
                     src1, dst1, ts1, td1, h1,
                     acc_sh, st0, sd0, sh0, st1, sd1, sh1):
        cid = lax.axis_index("c")
        sid = lax.axis_index("s")
        wid = sid * _NC + cid
        ebase = wid * ept
        rbase = sid * _ROWS_PER_SUB
        bufs = ((src0, dst0, ts0, td0, h0, st0, sd0, sh0),
                (src1, dst1, ts1, td1, h1, st1, sd1, sh1))

        def zero_own_rows():
            def zrow(r, _):
                for j in range(_D // 16):
                    ts0[r, pl.ds(j * 16, 16)] = jnp.zeros((16,), jnp.float32)
                return 0
            lax.fori_loop(0, _CH, zrow, 0)
            for off, nr in _ROW_PIECES:
                pltpu.sync_copy(ts0.at[pl.ds(0, nr), :],
                                acc_sh.at[pl.ds(rbase + off, nr), :])

        def writeback(out_ref):
            for off, nr in _ROW_PIECES:
                pltpu.sync_copy(acc_sh.at[pl.ds(rbase + off, nr), :],
                                h0.at[pl.ds(0, nr), :])
                pltpu.sync_copy(h0.at[pl.ds(0, nr), :],
                                out_ref.at[cid, pl.ds(rbase + off, nr), :])

        def edge_sweep(with_h):
            def load_and_issue(b, ci):
                srcv, dstv, tsv, tdv, hv, st, sd_, sh = b
                base = ebase + ci * _CH
                pltpu.sync_copy(src_hbm.at[pl.ds(base, _CH)], srcv)
                pltpu.sync_copy(dst_hbm.at[pl.ds(base, _CH)], dstv)
                pltpu.async_copy(ts_hbm.at[srcv], tsv, st)
                pltpu.async_copy(td_hbm.at[dstv], tdv, sd_)
                if with_h:
                    pltpu.async_copy(h_hbm.at[srcv], hv, sh)

            def process(b):
                srcv, dstv, tsv, tdv, hv, st, sd_, sh = b
                pltpu.make_async_copy(ts_hbm.at[srcv], tsv, st).wait()
                pltpu.make_async_copy(td_hbm.at[dstv], tdv, sd_).wait()
                if with_h:
                    pltpu.make_async_copy(h_hbm.at[srcv], hv, sh).wait()

                def edge_body(e, _):
                    for j in range(_D // 16):
                        sl = pl.ds(j * 16, 16)
                        p = _edge_weight(tsv[e, sl], tdv[e, sl])
                        if with_h:
                            p = p * hv[e, sl]
                        hv[e, sl] = p
                    return 0
                lax.fori_loop(0, _CH, edge_body, 0)
                pltpu.sync_copy(hv, acc_sh.at[dstv], add=True)

            load_and_issue(bufs[0], 0)
            load_and_issue(bufs[1], 1)

            def pair(k, _):
                for sidx in (0, 1):
                    b = bufs[sidx]
                    process(b)

                    @pl.when(k < npair - 1)
                    def _():
                        load_and_issue(b, 2 * (k + 1) + sidx)
                return 0
            lax.fori_loop(0, npair, pair, 0)

        zero_own_rows()
        plsc.subcore_barrier()
        edge_sweep(with_h=True)
        plsc.subcore_barrier()
        writeback(acc_out)
        zero_own_rows()
        plsc.subcore_barrier()
        edge_sweep(with_h=False)
        plsc.subcore_barrier()
        writeback(den_out)

    return sc_edge_pass


def _expand_table(a):
    heads, ch = a.shape
    if heads == 8:
        s = (a[:, :, None, None]
             * jnp.eye(8, dtype=a.dtype)[:, None, :, None]
             * jnp.ones((1, 1, 1, 16), a.dtype))
        return s.reshape(128, 128)
    return jnp.tile(a.reshape(128, 1), (1, 128))


def kernel(x, edge_attr, W1, as1, ad1, b1, W2, as2, ad2, b2, W3, as3, ad3, b3,
           W4, as4, ad4, b4, W5, as5, ad5, b5, linW, linb, edge_index):
    n = x.shape[0]
    e = edge_index.shape[1]
    e_total = e + n
    ept = -(-e_total // _NW)
    ept = -(-ept // (2 * _CH)) * (2 * _CH)
    epad = ept * _NW

    ei = edge_index.astype(jnp.int32)
    loop = jnp.arange(n, dtype=jnp.int32)
    padv = jnp.full((epad - e_total,), n, jnp.int32)
    src = jnp.concatenate([ei[0], loop, padv])
    dst = jnp.concatenate([ei[1], loop, padv])

    xp = jnp.pad(x, ((0, _NPAD - n), (0, 0)))

    sc_edge = _make_sc_edge_pass(ept)

    h, t_s, t_d = _tc_first(xp, W1, _expand_table(as1), _expand_table(ad1))
    acc, den = sc_edge(h, t_s, t_d, src, dst)
    for (W, a_s, a_d, b) in ((W2, as2, ad2, b1), (W3, as3, ad3, b2),
                             (W4, as4, ad4, b3), (W5, as5, ad5, b4)):
        h, t_s, t_d = _tc_mid(acc[0], acc[1], den[0], den[1],
                              b.reshape(1, _D), W, _expand_table(a_s),
                              _expand_table(a_d))
        acc, den = sc_edge(h, t_s, t_d, src, dst)
    out = _tc_last(acc[0], acc[1], den[0], den[1], b5.reshape(1, _D),
                   linW, linb.reshape(1, _D))
    return out[:n]


# speedup vs baseline: 4.5074x; 1.1728x over previous
import functools

import jax
import jax.numpy as jnp
from jax import lax
from jax.experimental import pallas as pl
from jax.experimental.pallas import tpu as pltpu
from jax.experimental.pallas import tpu_sc as plsc

_N = 10000
_NPAD = 10112
_D = 128
_NC = 2
_NS = 16
_NW = _NC * _NS
_CH = 64
_ROWS_PER_SUB = _NPAD // _NS
_ROW_PIECES = [(i * 64, 64) for i in range(9)] + [(576, 56)]
_EPS = 1e-16
_SLOPE = 0.2


def _tc_first_body(x_ref, w_ref, ss_ref, sd_ref, h_ref, ts_ref, td_ref):
    h = jnp.dot(x_ref[...], w_ref[...], preferred_element_type=jnp.float32)
    h_ref[...] = h
    ts_ref[...] = jnp.dot(h, ss_ref[...], preferred_element_type=jnp.float32)
    td_ref[...] = jnp.dot(h, sd_ref[...], preferred_element_type=jnp.float32)


def _epilogue(acc0, acc1, den0, den1, b):
    f = (acc0 + acc1) / (den0 + den1 + _EPS) + b
    return jnp.maximum(f, 0.0)


def _tc_mid_body(a0_ref, a1_ref, d0_ref, d1_ref, b_ref, w_ref, ss_ref, sd_ref,
                 h_ref, ts_ref, td_ref):
    f = _epilogue(a0_ref[...], a1_ref[...], d0_ref[...], d1_ref[...], b_ref[...])
    h = jnp.dot(f, w_ref[...], preferred_element_type=jnp.float32)
    h_ref[...] = h
    ts_ref[...] = jnp.dot(h, ss_ref[...], preferred_element_type=jnp.float32)
    td_ref[...] = jnp.dot(h, sd_ref[...], preferred_element_type=jnp.float32)


def _tc_last_body(a0_ref, a1_ref, d0_ref, d1_ref, b_ref, w_ref, lb_ref, o_ref):
    f = _epilogue(a0_ref[...], a1_ref[...], d0_ref[...], d1_ref[...], b_ref[...])
    o = jnp.dot(f, w_ref[...], preferred_element_type=jnp.float32) + lb_ref[...]
    o_ref[...] = jnp.maximum(o, 0.0)


def _row_spec():
    return pl.BlockSpec((_NPAD, _D), lambda: (0, 0))


def _full_spec(r):
    return pl.BlockSpec((r, _D), lambda: (0, 0))


_3OUT = [jax.ShapeDtypeStruct((_NPAD, _D), jnp.float32)] * 3

_tc_first = pl.pallas_call(
    _tc_first_body,
    in_specs=[_row_spec(), _full_spec(_D), _full_spec(_D), _full_spec(_D)],
    out_specs=[_row_spec()] * 3,
    out_shape=_3OUT,
)

_tc_mid = pl.pallas_call(
    _tc_mid_body,
    in_specs=[_row_spec(), _row_spec(), _row_spec(), _row_spec(),
              _full_spec(1), _full_spec(_D), _full_spec(_D), _full_spec(_D)],
    out_specs=[_row_spec()] * 3,
    out_shape=_3OUT,
)

_tc_last = pl.pallas_call(
    _tc_last_body,
    in_specs=[_row_spec(), _row_spec(), _row_spec(), _row_spec(),
              _full_spec(1), _full_spec(_D), _full_spec(1)],
    out_specs=_row_spec(),
    out_shape=jax.ShapeDtypeStruct((_NPAD, _D), jnp.float32),
)


def _edge_weight(s16, d16):
    er = s16 + d16
    er = jnp.where(er >= 0.0, er, er * _SLOPE)
    return jnp.exp(er)


def _make_sc_edge_pass(ept):
    nchunk = ept // _CH
    npair = nchunk // 2
    mesh = plsc.VectorSubcoreMesh(core_axis_name="c", subcore_axis_name="s")

    @functools.partial(
        pl.kernel,
        out_type=(jax.ShapeDtypeStruct((_NC, _NPAD, _D), jnp.float32),
                  jax.ShapeDtypeStruct((_NC, _NPAD, _D), jnp.float32)),
        mesh=mesh,
        scratch_types=[
            pltpu.VMEM((2, _CH), jnp.int32),
            pltpu.VMEM((_CH,), jnp.int32),
            pltpu.VMEM((_CH, _D), jnp.float32),
            pltpu.VMEM((_CH, _D), jnp.float32),
            pltpu.VMEM((_CH, _D), jnp.float32),
            pltpu.VMEM((2, _CH), jnp.int32),
            pltpu.VMEM((_CH,), jnp.int32),
            pltpu.VMEM((_CH, _D), jnp.float32),
            pltpu.VMEM((_CH, _D), jnp.float32),
            pltpu.VMEM((_CH, _D), jnp.float32),
            pltpu.VMEM_SHARED((_NPAD, _D), jnp.float32),
            pltpu.SemaphoreType.DMA,
            pltpu.SemaphoreType.DMA,
            pltpu.SemaphoreType.DMA,
            pltpu.SemaphoreType.DMA,
            pltpu.SemaphoreType.DMA,
            pltpu.SemaphoreType.DMA,
        ],
    )
    def sc_edge_pass(h_hbm, ts_hbm, td_hbm, sd_hbm,
                     acc_out, den_out,
                     sdv0, dstv0, ts0, td0, h0,
                     sdv1, dstv1, ts1, td1, h1,
                     acc_sh, st0, sd0, sh0, st1, sd1, sh1):
        cid = lax.axis_index("c")
        sid = lax.axis_index("s")
        wid = sid * _NC + cid
        cbase = wid * nchunk
        rbase = sid * _ROWS_PER_SUB
        bufs = ((sdv0, dstv0, ts0, td0, h0, st0, sd0, sh0),
                (sdv1, dstv1, ts1, td1, h1, st1, sd1, sh1))

        def zero_own_rows():
            def zrow(r, _):
                for j in range(_D // 16):
                    ts0[r, pl.ds(j * 16, 16)] = jnp.zeros((16,), jnp.float32)
                return 0
            lax.fori_loop(0, _CH, zrow, 0)
            for off, nr in _ROW_PIECES:
                pltpu.sync_copy(ts0.at[pl.ds(0, nr), :],
                                acc_sh.at[pl.ds(rbase + off, nr), :])

        def writeback(out_ref):
            for off, nr in _ROW_PIECES:
                pltpu.sync_copy(acc_sh.at[pl.ds(rbase + off, nr), :],
                                h0.at[pl.ds(0, nr), :])
                pltpu.sync_copy(h0.at[pl.ds(0, nr), :],
                                out_ref.at[cid, pl.ds(rbase + off, nr), :])

        def edge_sweep(with_h):
            def load_and_issue(b, ci):
                sdv, dstv, tsv, tdv, hv, st, sd_, sh = b
                pltpu.sync_copy(sd_hbm.at[cbase + ci], sdv)
                for r in range(_CH // 16):
                    sl = pl.ds(r * 16, 16)
                    dstv[sl] = sdv[1, sl]
                pltpu.async_copy(ts_hbm.at[sdv.at[0]], tsv, st)
                pltpu.async_copy(td_hbm.at[dstv], tdv, sd_)
                if with_h:
                    pltpu.async_copy(h_hbm.at[sdv.at[0]], hv, sh)

            def process(b):
                sdv, dstv, tsv, tdv, hv, st, sd_, sh = b
                pltpu.make_async_copy(ts_hbm.at[sdv.at[0]], tsv, st).wait()
                pltpu.make_async_copy(td_hbm.at[dstv], tdv, sd_).wait()
                if with_h:
                    pltpu.make_async_copy(h_hbm.at[sdv.at[0]], hv, sh).wait()

                def edge_body(e, _):
                    for j in range(_D // 16):
                        sl = pl.ds(j * 16, 16)
                        p = _edge_weight(tsv[e, sl], tdv[e, sl])
                        if with_h:
                            p = p * hv[e, sl]
                        hv[e, sl] = p
                    return 0
                lax.fori_loop(0, _CH, edge_body, 0)
                pltpu.sync_copy(hv, acc_sh.at[dstv], add=True)

            load_and_issue(bufs[0], 0)
            load_and_issue(bufs[1], 1)

            def pair(k, _):
                for sidx in (0, 1):
                    b = bufs[sidx]
                    process(b)

                    @pl.when(k < npair - 1)
                    def _():
                        load_and_issue(b, 2 * (k + 1) + sidx)
                return 0
            lax.fori_loop(0, npair, pair, 0)

        zero_own_rows()
        plsc.subcore_barrier()
        edge_sweep(with_h=True)
        plsc.subcore_barrier()
        writeback(acc_out)
        zero_own_rows()
        plsc.subcore_barrier()
        edge_sweep(with_h=False)
        plsc.subcore_barrier()
        writeback(den_out)

    return sc_edge_pass


def _expand_table(a):
    heads, ch = a.shape
    if heads == 8:
        s = (a[:, :, None, None]
             * jnp.eye(8, dtype=a.dtype)[:, None, :, None]
             * jnp.ones((1, 1, 1, 16), a.dtype))
        return s.reshape(128, 128)
    return jnp.tile(a.reshape(128, 1), (1, 128))


def kernel(x, edge_attr, W1, as1, ad1, b1, W2, as2, ad2, b2, W3, as3, ad3, b3,
           W4, as4, ad4, b4, W5, as5, ad5, b5, linW, linb, edge_index):
    n = x.shape[0]
    e = edge_index.shape[1]
    e_total = e + n
    ept = -(-e_total // _NW)
    ept = -(-ept // (2 * _CH)) * (2 * _CH)
    epad = ept * _NW

    ei = edge_index.astype(jnp.int32)
    loop = jnp.arange(n, dtype=jnp.int32)
    padv = jnp.full((epad - e_total,), n, jnp.int32)
    src = jnp.concatenate([ei[0], loop, padv])
    dst = jnp.concatenate([ei[1], loop, padv])
    sd = jnp.stack([src.reshape(-1, _CH), dst.reshape(-1, _CH)], axis=1)

    xp = jnp.pad(x, ((0, _NPAD - n), (0, 0)))

    sc_edge = _make_sc_edge_pass(ept)

    h, t_s, t_d = _tc_first(xp, W1, _expand_table(as1), _expand_table(ad1))
    acc, den = sc_edge(h, t_s, t_d, sd)
    for (W, a_s, a_d, b) in ((W2, as2, ad2, b1), (W3, as3, ad3, b2),
                             (W4, as4, ad4, b3), (W5, as5, ad5, b4)):
        h, t_s, t_d = _tc_mid(acc[0], acc[1], den[0], den[1],
                              b.reshape(1, _D), W, _expand_table(a_s),
                              _expand_table(a_d))
        acc, den = sc_edge(h, t_s, t_d, sd)
    out = _tc_last(acc[0], acc[1], den[0], den[1], b5.reshape(1, _D),
                   linW, linb.reshape(1, _D))
    return out[:n]
